# Initial kernel scaffold; baseline (speedup 1.0000x reference)
import jax
import jax.numpy as jnp
from jax.experimental import pallas as pl


def kernel(class_logits, embedding):
    raise NotImplementedError("write your pallas kernel here")


# speedup vs baseline: 1.3477x; 1.3477x over previous
import jax
import jax.numpy as jnp
from jax import lax
from jax.experimental import pallas as pl


def _argmax_gather_body(x_ref, emb_ref, out_ref):
    x = x_ref[...]
    c = x.shape[1]
    cols = lax.broadcasted_iota(jnp.int32, x.shape, 1)
    xm = jnp.where(cols < c, x, -jnp.inf)
    m = jnp.max(xm, axis=1, keepdims=True)
    idx = jnp.min(jnp.where(xm == m, cols, c), axis=1)
    onehot = (cols == idx[:, None])
    out_ref[...] = jnp.dot(onehot.astype(jnp.float32), emb_ref[...],
                           preferred_element_type=jnp.float32)


def kernel(class_logits, embedding):
    n, c = class_logits.shape
    _, d = embedding.shape
    br = 512
    return pl.pallas_call(
        _argmax_gather_body,
        grid=(n // br,),
        in_specs=[
            pl.BlockSpec((br, c), lambda i: (i, 0)),
            pl.BlockSpec((c, d), lambda i: (0, 0)),
        ],
        out_specs=pl.BlockSpec((br, d), lambda i: (i, 0)),
        out_shape=jax.ShapeDtypeStruct((n, d), jnp.float32),
    )(class_logits, embedding)
